# baseline (speedup 1.0000x reference)
import jax
import jax.numpy as jnp
from jax import lax
from jax.experimental import pallas as pl
from jax.experimental.pallas import tpu as pltpu
from jax.experimental.pallas import tpu_sc as plsc

B = 16384
VOCAB = 1000
VPAD = 1024
DIM = 64

SC_B = 8192
TC_B = B - SC_B
TBLK = 512

NUM_CORES = 2
NUM_SUBCORES = 16
NW = NUM_CORES * NUM_SUBCORES
BPW = SC_B // NW
CHUNK = 128
NCHUNK = BPW // CHUNK
LANES = 16
GPC = CHUNK // LANES
DSTEP = 8


def _sc_body(user_hbm, item_hbm, uf_hbm, itf_hbm, umap_hbm, imap_hbm,
             out_hbm, uidx_v, iidx_v, umap_v, imap_v, urows_v, irows_v,
             out_v, uf_s, itf_s, sems):
    sid = lax.axis_index("s")
    wid = sid * NUM_CORES + lax.axis_index("c")
    base = pl.multiple_of(wid * BPW, BPW)

    half = NUM_SUBCORES // 2
    trows = VOCAB // half
    toff = pl.multiple_of((sid % half) * trows, trows)
    tsl = pl.ds(toff, trows)

    stage = [
        pltpu.async_copy(user_hbm.at[pl.ds(base, BPW)], uidx_v, sems[0]),
        pltpu.async_copy(item_hbm.at[pl.ds(base, BPW)], iidx_v, sems[1]),
        pltpu.async_copy(umap_hbm, umap_v, sems[2]),
        pltpu.async_copy(imap_hbm, imap_v, sems[3]),
    ]

    @pl.when(sid < half)
    def _():
        pltpu.async_copy(uf_hbm.at[tsl], uf_s.at[tsl],
                         sems[4 + 2 * NCHUNK]).wait()

    @pl.when(sid >= half)
    def _():
        pltpu.async_copy(itf_hbm.at[tsl], itf_s.at[tsl],
                         sems[4 + 2 * NCHUNK]).wait()

    for c in stage:
        c.wait()

    copies = []
    for j in range(NCHUNK):
        def map_body(i, carry):
            off = pl.multiple_of((j * GPC + i) * LANES, LANES)
            raw_u = uidx_v[pl.ds(off, LANES)]
            raw_i = iidx_v[pl.ds(off, LANES)]
            uidx_v[pl.ds(off, LANES)] = plsc.load_gather(umap_v, [raw_u])
            iidx_v[pl.ds(off, LANES)] = plsc.load_gather(imap_v, [raw_i])
            return carry
        lax.fori_loop(0, GPC, map_body, 0)
        if j == 0:
            plsc.subcore_barrier()
        sl = pl.ds(j * CHUNK, CHUNK)
        copies.append(pltpu.async_copy(
            uf_s.at[uidx_v.at[sl]], urows_v.at[sl], sems[4 + 2 * j]))
        copies.append(pltpu.async_copy(
            itf_s.at[iidx_v.at[sl]], irows_v.at[sl], sems[5 + 2 * j]))

    lane = lax.iota(jnp.int32, LANES)
    outs = []
    for j in range(NCHUNK):
        copies[2 * j].wait()
        copies[2 * j + 1].wait()

        def group(g, carry):
            row = (j * GPC + g) * LANES + lane

            def dblock(db, st):
                acc, col = st
                for _ in range(DSTEP):
                    u = plsc.load_gather(urows_v, [row, col])
                    t = plsc.load_gather(irows_v, [row, col])
                    acc = acc + u * t
                    col = (col + 1) & (DIM - 1)
                return acc, col

            acc, _ = lax.fori_loop(
                0, DIM // DSTEP, dblock,
                (jnp.zeros((LANES,), jnp.float32), lane))
            off = pl.multiple_of((j * GPC + g) * LANES, LANES)
            out_v[pl.ds(off, LANES)] = acc
            return carry
        lax.fori_loop(0, GPC, group, 0)
        osl = pl.ds(j * CHUNK, CHUNK)
        outs.append(pltpu.async_copy(
            out_v.at[osl], out_hbm.at[pl.ds(base + j * CHUNK, CHUNK)],
            sems[5 + 2 * NCHUNK + j]))

    for o in outs:
        o.wait()


def _sc_part(user, item, user_factors, item_factors, user_map, item_map):
    mesh = plsc.VectorSubcoreMesh(
        core_axis_name="c", subcore_axis_name="s",
        num_cores=NUM_CORES, num_subcores=NUM_SUBCORES)
    return pl.kernel(
        _sc_body,
        out_type=jax.ShapeDtypeStruct((SC_B,), jnp.float32),
        mesh=mesh,
        scratch_types=[
            pltpu.VMEM((BPW,), jnp.int32),
            pltpu.VMEM((BPW,), jnp.int32),
            pltpu.VMEM((VOCAB,), jnp.int32),
            pltpu.VMEM((VOCAB,), jnp.int32),
            pltpu.VMEM((BPW, DIM), jnp.float32),
            pltpu.VMEM((BPW, DIM), jnp.float32),
            pltpu.VMEM((BPW,), jnp.float32),
            pltpu.VMEM_SHARED((VOCAB, DIM), jnp.float32),
            pltpu.VMEM_SHARED((VOCAB, DIM), jnp.float32),
            [pltpu.SemaphoreType.DMA] * (5 + 3 * NCHUNK),
        ],
        compiler_params=pltpu.CompilerParams(
            needs_layout_passes=False, use_tc_tiling_on_sc=False),
    )(user, item, user_factors, item_factors, user_map, item_map)


def _tc_body(user_ref, item_ref, uf_ref, itf_ref, umap_ref, imap_ref,
             out_ref, ufm_ref, itfm_ref):
    @pl.when(pl.program_id(0) == 0)
    def _():
        rows = lax.broadcasted_iota(jnp.int32, (VPAD, VPAD), 1)
        moh_u = (rows == umap_ref[...][:, None]).astype(jnp.bfloat16)
        moh_i = (rows == imap_ref[...][:, None]).astype(jnp.bfloat16)
        ufm_ref[...] = jnp.dot(
            moh_u, uf_ref[...], preferred_element_type=jnp.float32
        ).astype(jnp.bfloat16)
        itfm_ref[...] = jnp.dot(
            moh_i, itf_ref[...], preferred_element_type=jnp.float32
        ).astype(jnp.bfloat16)

    u = user_ref[0, 0, :]
    it = item_ref[0, 0, :]
    vlane = lax.broadcasted_iota(jnp.int32, (TBLK, VPAD), 1)
    oh_u = (vlane == u[:, None]).astype(jnp.bfloat16)
    oh_i = (vlane == it[:, None]).astype(jnp.bfloat16)
    uf_rows = jnp.dot(oh_u, ufm_ref[...],
                      preferred_element_type=jnp.float32)
    itf_rows = jnp.dot(oh_i, itfm_ref[...],
                       preferred_element_type=jnp.float32)
    out_ref[0, 0, :] = jnp.sum(uf_rows * itf_rows, axis=1)


def _tc_part(user, item, uf, itf, umap, imap):
    nblk = TC_B // TBLK
    uf_b = jnp.pad(uf, ((0, VPAD - VOCAB), (0, 0))).astype(jnp.bfloat16)
    itf_b = jnp.pad(itf, ((0, VPAD - VOCAB), (0, 0))).astype(jnp.bfloat16)
    umap_p = jnp.pad(umap, (0, VPAD - VOCAB), constant_values=VPAD - 1)
    imap_p = jnp.pad(imap, (0, VPAD - VOCAB), constant_values=VPAD - 1)
    return pl.pallas_call(
        _tc_body,
        grid=(nblk,),
        in_specs=[
            pl.BlockSpec((1, 1, TBLK), lambda i: (i, 0, 0)),
            pl.BlockSpec((1, 1, TBLK), lambda i: (i, 0, 0)),
            pl.BlockSpec((VPAD, DIM), lambda i: (0, 0)),
            pl.BlockSpec((VPAD, DIM), lambda i: (0, 0)),
            pl.BlockSpec((VPAD,), lambda i: (0,)),
            pl.BlockSpec((VPAD,), lambda i: (0,)),
        ],
        out_specs=pl.BlockSpec((1, 1, TBLK), lambda i: (i, 0, 0)),
        out_shape=jax.ShapeDtypeStruct((nblk, 1, TBLK), jnp.float32),
        scratch_shapes=[
            pltpu.VMEM((VPAD, DIM), jnp.bfloat16),
            pltpu.VMEM((VPAD, DIM), jnp.bfloat16),
        ],
    )(user.reshape(nblk, 1, TBLK), item.reshape(nblk, 1, TBLK),
      uf_b, itf_b, umap_p, imap_p).reshape(TC_B)


@jax.jit
def _svd_dot(user, item, user_factors, item_factors, user_map, item_map):
    sc_out = _sc_part(user[:SC_B], item[:SC_B], user_factors,
                      item_factors, user_map, item_map)
    tc_out = _tc_part(user[SC_B:], item[SC_B:], user_factors,
                      item_factors, user_map, item_map)
    return jnp.concatenate([sc_out, tc_out])


def kernel(user, item, user_factors, item_factors, user_map, item_map):
    return _svd_dot(user, item, user_factors, item_factors,
                    user_map, item_map)


# speedup vs baseline: 1.8857x; 1.8857x over previous
import jax
import jax.numpy as jnp
from jax import lax
from jax.experimental import pallas as pl
from jax.experimental.pallas import tpu as pltpu
from jax.experimental.pallas import tpu_sc as plsc

B = 16384
VOCAB = 1000
DIM = 64

NUM_CORES = 2
NUM_SUBCORES = 16
NW = NUM_CORES * NUM_SUBCORES
BPW = B // NW
CHUNK = 128
NCHUNK = BPW // CHUNK
LANES = 16
GPC = CHUNK // LANES
DSTEP = 8


def _body(user_hbm, item_hbm, uf_hbm, itf_hbm, umap_hbm, imap_hbm,
          out_hbm, uidx_v, iidx_v, umap_v, imap_v, urows_v, irows_v,
          out_v, uf_s, itf_s, sems):
    sid = lax.axis_index("s")
    wid = sid * NUM_CORES + lax.axis_index("c")
    base = pl.multiple_of(wid * BPW, BPW)

    half = NUM_SUBCORES // 2
    trows = VOCAB // half
    toff = pl.multiple_of((sid % half) * trows, trows)
    tsl = pl.ds(toff, trows)

    stage = [
        pltpu.async_copy(user_hbm.at[pl.ds(base, BPW)], uidx_v, sems[0]),
        pltpu.async_copy(item_hbm.at[pl.ds(base, BPW)], iidx_v, sems[1]),
        pltpu.async_copy(umap_hbm, umap_v, sems[2]),
        pltpu.async_copy(imap_hbm, imap_v, sems[3]),
    ]

    @pl.when(sid < half)
    def _():
        pltpu.async_copy(uf_hbm.at[tsl], uf_s.at[tsl],
                         sems[4 + 2 * NCHUNK]).wait()

    @pl.when(sid >= half)
    def _():
        pltpu.async_copy(itf_hbm.at[tsl], itf_s.at[tsl],
                         sems[4 + 2 * NCHUNK]).wait()

    for c in stage:
        c.wait()

    copies = []
    for j in range(NCHUNK):
        def map_body(i, carry):
            off = pl.multiple_of((j * GPC + i) * LANES, LANES)
            raw_u = uidx_v[pl.ds(off, LANES)]
            raw_i = iidx_v[pl.ds(off, LANES)]
            uidx_v[pl.ds(off, LANES)] = plsc.load_gather(umap_v, [raw_u])
            iidx_v[pl.ds(off, LANES)] = plsc.load_gather(imap_v, [raw_i])
            return carry
        lax.fori_loop(0, GPC, map_body, 0)
        if j == 0:
            plsc.subcore_barrier()
        sl = pl.ds(j * CHUNK, CHUNK)
        copies.append(pltpu.async_copy(
            uf_s.at[uidx_v.at[sl]], urows_v.at[sl], sems[4 + 2 * j]))
        copies.append(pltpu.async_copy(
            itf_s.at[iidx_v.at[sl]], irows_v.at[sl], sems[5 + 2 * j]))

    lane = lax.iota(jnp.int32, LANES)
    outs = []
    for j in range(NCHUNK):
        copies[2 * j].wait()
        copies[2 * j + 1].wait()

        def group(g, carry):
            row = (j * GPC + g) * LANES + lane

            def dblock(db, st):
                acc, col = st
                for _ in range(DSTEP):
                    u = plsc.load_gather(urows_v, [row, col])
                    t = plsc.load_gather(irows_v, [row, col])
                    acc = acc + u * t
                    col = (col + 1) & (DIM - 1)
                return acc, col

            acc, _ = lax.fori_loop(
                0, DIM // DSTEP, dblock,
                (jnp.zeros((LANES,), jnp.float32), lane))
            off = pl.multiple_of((j * GPC + g) * LANES, LANES)
            out_v[pl.ds(off, LANES)] = acc
            return carry
        lax.fori_loop(0, GPC, group, 0)
        osl = pl.ds(j * CHUNK, CHUNK)
        outs.append(pltpu.async_copy(
            out_v.at[osl], out_hbm.at[pl.ds(base + j * CHUNK, CHUNK)],
            sems[5 + 2 * NCHUNK + j]))

    for o in outs:
        o.wait()


@jax.jit
def _svd_dot(user, item, user_factors, item_factors, user_map, item_map):
    mesh = plsc.VectorSubcoreMesh(
        core_axis_name="c", subcore_axis_name="s",
        num_cores=NUM_CORES, num_subcores=NUM_SUBCORES)
    return pl.kernel(
        _body,
        out_type=jax.ShapeDtypeStruct((B,), jnp.float32),
        mesh=mesh,
        scratch_types=[
            pltpu.VMEM((BPW,), jnp.int32),
            pltpu.VMEM((BPW,), jnp.int32),
            pltpu.VMEM((VOCAB,), jnp.int32),
            pltpu.VMEM((VOCAB,), jnp.int32),
            pltpu.VMEM((BPW, DIM), jnp.float32),
            pltpu.VMEM((BPW, DIM), jnp.float32),
            pltpu.VMEM((BPW,), jnp.float32),
            pltpu.VMEM_SHARED((VOCAB, DIM), jnp.float32),
            pltpu.VMEM_SHARED((VOCAB, DIM), jnp.float32),
            [pltpu.SemaphoreType.DMA] * (5 + 3 * NCHUNK),
        ],
        compiler_params=pltpu.CompilerParams(
            needs_layout_passes=False, use_tc_tiling_on_sc=False),
    )(user, item, user_factors, item_factors, user_map, item_map)


def kernel(user, item, user_factors, item_factors, user_map, item_map):
    return _svd_dot(user, item, user_factors, item_factors,
                    user_map, item_map)
